# baseline (speedup 1.0000x reference)
import jax
import jax.numpy as jnp
from jax import lax
from jax.experimental import pallas as pl
from jax.experimental.pallas import tpu as pltpu
from jax.experimental.pallas import tpu_sc as plsc

N = 3_200_000
NUM_SEG = 16_384
NC = 2
NS = 16
L = 16
NW = NC * NS
CHUNK = N // NW
SLICE = 20_000
NUM_SLICES = CHUNK // SLICE

VECS = SLICE // L
U_SC = 10
NBUF = 2

EW_COLS = 128
EW_ROWS = N // EW_COLS
EW_BLOCK = 1000


def _sc_body(ea_hbm, seg_hbm, part_hbm,
             ea_v0, ea_v1, seg_v0, seg_v1,
             acc_v, in_sem0, in_sem1):
    wid = lax.axis_index("s") * NC + lax.axis_index("c")
    base = wid * CHUNK
    ea_bufs = (ea_v0, ea_v1)
    seg_bufs = (seg_v0, seg_v1)
    in_sems = (in_sem0, in_sem1)

    def zero_body(i, carry):
        for j in range(16):
            acc_v[pl.ds((i * 16 + j) * L, L)] = jnp.zeros((L,), jnp.float32)
        return carry
    lax.fori_loop(0, NUM_SEG // (16 * L), zero_body, 0)

    def start_in(s):
        off = base + s * SLICE
        b = s % NBUF
        copies = [
            pltpu.async_copy(ea_hbm.at[pl.ds(off, SLICE)], ea_bufs[b], in_sems[b]),
            pltpu.async_copy(seg_hbm.at[pl.ds(off, SLICE)],
                             seg_bufs[b].at[pl.ds(0, SLICE)], in_sems[b]),
        ]
        if s + 1 < NUM_SLICES:
            copies.append(pltpu.async_copy(
                seg_hbm.at[pl.ds(off + SLICE, L)],
                seg_bufs[b].at[pl.ds(SLICE, L)], in_sems[b]))
        return copies

    pending_in = {0: start_in(0)}
    run_vec = jnp.zeros((L,), jnp.float32)
    for s in range(NUM_SLICES):
        b = s % NBUF
        if s + 1 < NUM_SLICES:
            pending_in[s + 1] = start_in(s + 1)
        for d in pending_in.pop(s):
            d.wait()

        eab, segb = ea_bufs[b], seg_bufs[b]
        if s == NUM_SLICES - 1:
            segb[pl.ds(SLICE, L)] = jnp.full((L,), -1, jnp.int32)

        @plsc.parallel_loop(0, VECS, 1, unroll=U_SC, carry=run_vec)
        def vec_body(v, rv):
            sl = pl.ds(v * L, L)
            ea = eab[sl]
            idx = segb[sl]
            nxt = segb[pl.ds(v * L + 1, L)]
            t = plsc.cumsum(ea) + rv
            end = idx != nxt
            plsc.addupdate_scatter(acc_v, [idx], t, mask=end)
            plsc.addupdate_scatter(acc_v, [nxt], -t, mask=end & (nxt >= 0))
            return rv + jnp.sum(ea)
        run_vec = vec_body

    pltpu.sync_copy(acc_v, part_hbm.at[wid])


def _add_body(a_ref, b_ref, o_ref):
    o_ref[...] = a_ref[...] + b_ref[...]


def _combine_body(p_ref, e_ref):
    e_ref[...] = jnp.sum(p_ref[...], axis=0)


def kernel(Ea, E2a, Za, batch_seg):
    seg = batch_seg.astype(jnp.int32)
    mesh = plsc.VectorSubcoreMesh(core_axis_name="c", subcore_axis_name="s")
    sc = pl.kernel(
        _sc_body,
        out_type=jax.ShapeDtypeStruct((NW, NUM_SEG), jnp.float32),
        mesh=mesh,
        compiler_params=pltpu.CompilerParams(
            needs_layout_passes=False, disable_bounds_checks=True),
        scratch_types=[
            pltpu.VMEM((SLICE,), jnp.float32),
            pltpu.VMEM((SLICE,), jnp.float32),
            pltpu.VMEM((SLICE + L,), jnp.int32),
            pltpu.VMEM((SLICE + L,), jnp.int32),
            pltpu.VMEM((NUM_SEG,), jnp.float32),
            pltpu.SemaphoreType.DMA,
            pltpu.SemaphoreType.DMA,
        ],
    )
    partials = sc(Ea, seg)
    ea_out = pl.pallas_call(
        _add_body,
        grid=(EW_ROWS // EW_BLOCK,),
        in_specs=[pl.BlockSpec((EW_BLOCK, EW_COLS), lambda i: (i, 0))] * 2,
        out_specs=pl.BlockSpec((EW_BLOCK, EW_COLS), lambda i: (i, 0)),
        out_shape=jax.ShapeDtypeStruct((EW_ROWS, EW_COLS), jnp.float32),
    )(Ea.reshape(EW_ROWS, EW_COLS), E2a.reshape(EW_ROWS, EW_COLS))
    e = pl.pallas_call(
        _combine_body,
        out_shape=jax.ShapeDtypeStruct((NUM_SEG,), jnp.float32),
    )(partials)
    return ea_out.reshape(N), e


# speedup vs baseline: 1.0103x; 1.0103x over previous
import jax
import jax.numpy as jnp
from jax import lax
from jax.experimental import pallas as pl
from jax.experimental.pallas import tpu as pltpu
from jax.experimental.pallas import tpu_sc as plsc

N = 3_200_000
NUM_SEG = 16_384
NC = 2
NS = 16
L = 16
NW = NC * NS
CHUNK = N // NW
SLICE = 10_000
NUM_SLICES = CHUNK // SLICE

VECS = SLICE // L
U_SC = 25
NBUF = 2

EW_COLS = 128
EW_ROWS = N // EW_COLS
EW_BLOCK = 1000


def _sc_body(ea_hbm, seg_hbm, part_hbm,
             ea_v0, ea_v1, seg_v0, seg_v1,
             acc_v, in_sem0, in_sem1):
    wid = lax.axis_index("s") * NC + lax.axis_index("c")
    base = wid * CHUNK
    ea_bufs = (ea_v0, ea_v1)
    seg_bufs = (seg_v0, seg_v1)
    in_sems = (in_sem0, in_sem1)

    def zero_body(i, carry):
        for j in range(16):
            acc_v[pl.ds((i * 16 + j) * L, L)] = jnp.zeros((L,), jnp.float32)
        return carry
    lax.fori_loop(0, NUM_SEG // (16 * L), zero_body, 0)

    def start_in(s):
        off = base + s * SLICE
        b = s % NBUF
        copies = [
            pltpu.async_copy(ea_hbm.at[pl.ds(off, SLICE)], ea_bufs[b], in_sems[b]),
            pltpu.async_copy(seg_hbm.at[pl.ds(off, SLICE)],
                             seg_bufs[b].at[pl.ds(0, SLICE)], in_sems[b]),
        ]
        if s + 1 < NUM_SLICES:
            copies.append(pltpu.async_copy(
                seg_hbm.at[pl.ds(off + SLICE, L)],
                seg_bufs[b].at[pl.ds(SLICE, L)], in_sems[b]))
        return copies

    pending_in = {0: start_in(0)}
    run_vec = jnp.zeros((L,), jnp.float32)
    for s in range(NUM_SLICES):
        b = s % NBUF
        if s + 1 < NUM_SLICES:
            pending_in[s + 1] = start_in(s + 1)
        for d in pending_in.pop(s):
            d.wait()

        eab, segb = ea_bufs[b], seg_bufs[b]
        if s == NUM_SLICES - 1:
            segb[pl.ds(SLICE, L)] = jnp.full((L,), -1, jnp.int32)

        @plsc.parallel_loop(0, VECS, 1, unroll=U_SC, carry=run_vec)
        def vec_body(v, rv):
            sl = pl.ds(v * L, L)
            ea = eab[sl]
            idx = segb[sl]
            nxt = segb[pl.ds(v * L + 1, L)]
            t = plsc.cumsum(ea) + rv
            end = idx != nxt
            plsc.addupdate_scatter(acc_v, [idx], t, mask=end)
            plsc.addupdate_scatter(acc_v, [nxt], -t, mask=end & (nxt >= 0))
            return rv + jnp.sum(ea)
        run_vec = vec_body

    pltpu.sync_copy(acc_v, part_hbm.at[wid])


def _add_body(a_ref, b_ref, o_ref):
    o_ref[...] = a_ref[...] + b_ref[...]


def _combine_body(p_ref, e_ref):
    e_ref[...] = jnp.sum(p_ref[...], axis=0)


def kernel(Ea, E2a, Za, batch_seg):
    seg = batch_seg.astype(jnp.int32)
    mesh = plsc.VectorSubcoreMesh(core_axis_name="c", subcore_axis_name="s")
    sc = pl.kernel(
        _sc_body,
        out_type=jax.ShapeDtypeStruct((NW, NUM_SEG), jnp.float32),
        mesh=mesh,
        compiler_params=pltpu.CompilerParams(
            needs_layout_passes=False, disable_bounds_checks=True),
        scratch_types=[
            pltpu.VMEM((SLICE,), jnp.float32),
            pltpu.VMEM((SLICE,), jnp.float32),
            pltpu.VMEM((SLICE + L,), jnp.int32),
            pltpu.VMEM((SLICE + L,), jnp.int32),
            pltpu.VMEM((NUM_SEG,), jnp.float32),
            pltpu.SemaphoreType.DMA,
            pltpu.SemaphoreType.DMA,
        ],
    )
    partials = sc(Ea, seg)
    ea_out = pl.pallas_call(
        _add_body,
        grid=(EW_ROWS // EW_BLOCK,),
        in_specs=[pl.BlockSpec((EW_BLOCK, EW_COLS), lambda i: (i, 0))] * 2,
        out_specs=pl.BlockSpec((EW_BLOCK, EW_COLS), lambda i: (i, 0)),
        out_shape=jax.ShapeDtypeStruct((EW_ROWS, EW_COLS), jnp.float32),
    )(Ea.reshape(EW_ROWS, EW_COLS), E2a.reshape(EW_ROWS, EW_COLS))
    e = pl.pallas_call(
        _combine_body,
        out_shape=jax.ShapeDtypeStruct((NUM_SEG,), jnp.float32),
    )(partials)
    return ea_out.reshape(N), e


# speedup vs baseline: 1.0403x; 1.0297x over previous
import jax
import jax.numpy as jnp
from jax import lax
from jax.experimental import pallas as pl
from jax.experimental.pallas import tpu as pltpu
from jax.experimental.pallas import tpu_sc as plsc

N = 3_200_000
NUM_SEG = 16_384
NC = 2
NS = 16
L = 16
NW = NC * NS
CHUNK = N // NW
SLICE = 10_000
NUM_SLICES = CHUNK // SLICE

VECS = SLICE // L
U_SC = 5
NBUF = 2

EW_COLS = 128
EW_ROWS = N // EW_COLS
EW_BLOCK = 1000


def _sc_body(ea_hbm, seg_hbm, part_hbm,
             ea_v0, ea_v1, seg_v0, seg_v1,
             acc_v, in_sem0, in_sem1):
    wid = lax.axis_index("s") * NC + lax.axis_index("c")
    base = wid * CHUNK
    ea_bufs = (ea_v0, ea_v1)
    seg_bufs = (seg_v0, seg_v1)
    in_sems = (in_sem0, in_sem1)

    def zero_body(i, carry):
        for j in range(16):
            acc_v[pl.ds((i * 16 + j) * L, L)] = jnp.zeros((L,), jnp.float32)
        return carry
    lax.fori_loop(0, NUM_SEG // (16 * L), zero_body, 0)

    def start_in(s):
        off = base + s * SLICE
        b = s % NBUF
        copies = [
            pltpu.async_copy(ea_hbm.at[pl.ds(off, SLICE)], ea_bufs[b], in_sems[b]),
            pltpu.async_copy(seg_hbm.at[pl.ds(off, SLICE)],
                             seg_bufs[b].at[pl.ds(L, SLICE)], in_sems[b]),
        ]
        if s + 1 < NUM_SLICES:
            copies.append(pltpu.async_copy(
                seg_hbm.at[pl.ds(off + SLICE, L)],
                seg_bufs[b].at[pl.ds(L + SLICE, L)], in_sems[b]))
        if s > 0:
            copies.append(pltpu.async_copy(
                seg_hbm.at[pl.ds(off - L, L)],
                seg_bufs[b].at[pl.ds(0, L)], in_sems[b]))
        return copies

    pending_in = {0: start_in(0)}
    run_vec = jnp.zeros((L,), jnp.float32)
    for s in range(NUM_SLICES):
        b = s % NBUF
        if s + 1 < NUM_SLICES:
            pending_in[s + 1] = start_in(s + 1)
        for d in pending_in.pop(s):
            d.wait()

        eab, segb = ea_bufs[b], seg_bufs[b]
        if s == NUM_SLICES - 1:
            segb[pl.ds(L + SLICE, L)] = jnp.full((L,), -1, jnp.int32)
        if s == 0:
            segb[pl.ds(0, L)] = jnp.full((L,), -2, jnp.int32)

        @plsc.parallel_loop(0, VECS, 1, unroll=U_SC, carry=run_vec)
        def vec_body(v, rv):
            ea = eab[pl.ds(v * L, L)]
            idx = segb[pl.ds(L + v * L, L)]
            nxt = segb[pl.ds(L + v * L + 1, L)]
            prv = segb[pl.ds(L + v * L - 1, L)]
            t = plsc.cumsum(ea) + rv
            is_end = idx != nxt
            is_start = idx != prv
            val = (jnp.where(is_end, t, 0.0)
                   - jnp.where(is_start, t - ea, 0.0))
            plsc.addupdate_scatter(acc_v, [idx], val, mask=is_end | is_start)
            return rv + jnp.sum(ea)
        run_vec = vec_body

    pltpu.sync_copy(acc_v, part_hbm.at[wid])


def _add_body(a_ref, b_ref, o_ref):
    o_ref[...] = a_ref[...] + b_ref[...]


def _combine_body(p_ref, e_ref):
    e_ref[...] = jnp.sum(p_ref[...], axis=0)


def kernel(Ea, E2a, Za, batch_seg):
    seg = batch_seg.astype(jnp.int32)
    mesh = plsc.VectorSubcoreMesh(core_axis_name="c", subcore_axis_name="s")
    sc = pl.kernel(
        _sc_body,
        out_type=jax.ShapeDtypeStruct((NW, NUM_SEG), jnp.float32),
        mesh=mesh,
        compiler_params=pltpu.CompilerParams(
            needs_layout_passes=False, disable_bounds_checks=True),
        scratch_types=[
            pltpu.VMEM((SLICE,), jnp.float32),
            pltpu.VMEM((SLICE,), jnp.float32),
            pltpu.VMEM((SLICE + 2 * L,), jnp.int32),
            pltpu.VMEM((SLICE + 2 * L,), jnp.int32),
            pltpu.VMEM((NUM_SEG,), jnp.float32),
            pltpu.SemaphoreType.DMA,
            pltpu.SemaphoreType.DMA,
        ],
    )
    partials = sc(Ea, seg)
    ea_out = pl.pallas_call(
        _add_body,
        grid=(EW_ROWS // EW_BLOCK,),
        in_specs=[pl.BlockSpec((EW_BLOCK, EW_COLS), lambda i: (i, 0))] * 2,
        out_specs=pl.BlockSpec((EW_BLOCK, EW_COLS), lambda i: (i, 0)),
        out_shape=jax.ShapeDtypeStruct((EW_ROWS, EW_COLS), jnp.float32),
    )(Ea.reshape(EW_ROWS, EW_COLS), E2a.reshape(EW_ROWS, EW_COLS))
    e = pl.pallas_call(
        _combine_body,
        out_shape=jax.ShapeDtypeStruct((NUM_SEG,), jnp.float32),
    )(partials)
    return ea_out.reshape(N), e
